# Initial kernel scaffold; baseline (speedup 1.0000x reference)
import jax
import jax.numpy as jnp
from jax.experimental import pallas as pl


def kernel(x, Wg, bg, We, be):
    raise NotImplementedError("write your pallas kernel here")


# speedup vs baseline: 1.0735x; 1.0735x over previous
import functools

import jax
import jax.numpy as jnp
from jax.experimental import pallas as pl
from jax.experimental.pallas import tpu as pltpu

E = 8
K = 2
D = 2048
H = 2048
N = 2048
S = N * K
TM = 512
NT = S // TM
T = NT + E - 1


def _gate_kernel(x_ref, wg_ref, bg_ref, out_ref):
    x = x_ref[...]
    logits = jax.lax.dot_general(
        x, wg_ref[...], (((1,), (1,)), ((), ())),
        preferred_element_type=jnp.float32)
    logits = logits + bg_ref[...]
    m = jnp.max(logits, axis=-1, keepdims=True)
    ex = jnp.exp(logits - m)
    sm = ex / jnp.sum(ex, axis=-1, keepdims=True)
    iota = jax.lax.broadcasted_iota(jnp.int32, (N, E), 1).astype(jnp.float32)
    big = jnp.float32(E)
    m1 = jnp.max(sm, axis=-1, keepdims=True)
    i1 = jnp.min(jnp.where(sm == m1, iota, big), axis=-1, keepdims=True)
    sm2 = jnp.where(iota == i1, -jnp.float32(1.0), sm)
    m2 = jnp.max(sm2, axis=-1, keepdims=True)
    i2 = jnp.min(jnp.where(sm2 == m2, iota, big), axis=-1, keepdims=True)
    out_ref[...] = (m1 * (iota == 0) + m2 * (iota == 1)
                    + i1 * (iota == 2) + i2 * (iota == 3))


def _gate(x, Wg, bg):
    out = pl.pallas_call(
        _gate_kernel,
        out_shape=jax.ShapeDtypeStruct((N, E), jnp.float32),
    )(x, Wg, bg.reshape(1, E))
    scores = out[:, :K]
    idx = out[:, K:2 * K].astype(jnp.int32)
    return scores, idx


def _gmm_kernel(info_ref, xs_ref, we_ref, out_ref):
    t = pl.program_id(0)
    lo = info_ref[2, t]
    hi = info_ref[3, t]
    first = info_ref[4, t]
    rows = jax.lax.broadcasted_iota(jnp.int32, (TM, 1), 0)
    mask = (rows >= lo) & (rows < hi)
    x = jnp.where(mask, xs_ref[...], jnp.float32(0.0))
    contrib = jax.lax.dot_general(
        x, we_ref[0], (((1,), (1,)), ((), ())),
        preferred_element_type=jnp.float32)

    @pl.when(first == 1)
    def _():
        out_ref[...] = contrib

    @pl.when(first == 0)
    def _():
        out_ref[...] += contrib


def _grouped_matmul(info, x_sorted, We):
    grid_spec = pltpu.PrefetchScalarGridSpec(
        num_scalar_prefetch=1,
        grid=(T,),
        in_specs=[
            pl.BlockSpec((TM, D), lambda t, info: (info[0, t], 0)),
            pl.BlockSpec((1, H, D), lambda t, info: (info[1, t], 0, 0)),
        ],
        out_specs=pl.BlockSpec((TM, H), lambda t, info: (info[0, t], 0)),
    )
    return pl.pallas_call(
        _gmm_kernel,
        grid_spec=grid_spec,
        out_shape=jax.ShapeDtypeStruct((S, H), jnp.float32),
    )(info, x_sorted, We)


def _visit_maps(counts):
    offsets = jnp.concatenate(
        [jnp.zeros((1,), jnp.int32), jnp.cumsum(counts, dtype=jnp.int32)])
    first_tile = offsets[:E] // TM
    last_tile = jnp.maximum(offsets[1:] - 1, 0) // TM
    tiles_g = jnp.where(counts > 0, last_tile - first_tile + 1, 0)
    vb = jnp.concatenate(
        [jnp.zeros((1,), jnp.int32), jnp.cumsum(tiles_g, dtype=jnp.int32)])
    t_act = vb[E]
    tt = jnp.arange(T, dtype=jnp.int32)
    gid = jnp.sum((tt[:, None] >= vb[None, 1:]).astype(jnp.int32), axis=1)
    gid = jnp.clip(gid, 0, E - 1)
    valid = tt < t_act
    mt = first_tile[gid] + (tt - vb[gid])
    mt = jnp.where(valid, mt, NT - 1)
    glo = offsets[gid]
    ghi = offsets[gid + 1]
    lo = jnp.where(valid, jnp.clip(glo - mt * TM, 0, TM), 0)
    hi = jnp.where(valid, jnp.clip(ghi - mt * TM, 0, TM), 0)
    prev_mt = jnp.concatenate([jnp.full((1,), -1, jnp.int32), mt[:-1]])
    first = (valid & (mt != prev_mt)).astype(jnp.int32)
    return jnp.stack([mt, gid, lo, hi, first])


def kernel(x, Wg, bg, We, be):
    scores, idx = _gate(x, Wg, bg)

    e_slots = idx.reshape(-1)
    counts = jnp.sum(e_slots[None, :] == jnp.arange(E, dtype=jnp.int32)[:, None],
                     axis=1).astype(jnp.int32)
    order = jnp.argsort(e_slots, stable=True)
    pos = jnp.argsort(order)
    info = _visit_maps(counts)

    x_sorted = x[order // K]
    y_sorted = _grouped_matmul(info, x_sorted, We)

    y = y_sorted[pos].reshape(N, K, H)
    out = jnp.sum(y * scores[:, :, None], axis=1)
    out = out + jnp.einsum('nk,nkh->nh', scores, be[idx])
    return out
